# baseline (speedup 1.0000x reference)
import jax
import jax.numpy as jnp
from jax.experimental import pallas as pl
from jax.experimental.pallas import tpu as pltpu

HIDDEN = 2048
NUM_EXPERTS = 64
LOAD_BALANCING_LAMBDA = 0.01
TOKENS = 4 * 4096
BLOCK_T = 2048
HALF_T = BLOCK_T // 2
N_STEPS = TOKENS // BLOCK_T


def _half(x32, w2, wh):
    xh = x32.astype(jnp.bfloat16)
    xl = (x32 - xh.astype(jnp.float32)).astype(jnp.bfloat16)
    p1 = jax.lax.dot_general(
        xh, w2, (((1,), (1,)), ((), ())), preferred_element_type=jnp.float32)
    p2 = jax.lax.dot_general(
        xl, wh, (((1,), (1,)), ((), ())), preferred_element_type=jnp.float32)
    logits = (p1[:, 0:NUM_EXPERTS] + p1[:, NUM_EXPERTS:2 * NUM_EXPERTS]) + p2

    m = jnp.max(logits, axis=-1, keepdims=True)
    e = jnp.exp(logits - m)
    s = jnp.sum(e, axis=-1, keepdims=True)
    probs = e * (1.0 / s)

    iota = jax.lax.broadcasted_iota(jnp.int32, logits.shape, 1)
    eq = logits == m
    idx = jnp.min(jnp.where(eq, iota, NUM_EXPERTS), axis=-1, keepdims=True)
    mask = jnp.where(iota == idx, 1.0, 0.0)

    psum = jnp.sum(probs, axis=0, keepdims=True)
    usum = jnp.sum(mask, axis=0, keepdims=True)
    return logits, mask, psum, usum


def _router_kernel(x_ref, w2_ref, wh_ref, logits_ref, mask_ref, loss_ref,
                   psum_ref, usum_ref):
    i = pl.program_id(0)
    w2 = w2_ref[...]
    wh = wh_ref[...]

    l0, k0, p0, u0 = _half(x_ref[0:HALF_T, :], w2, wh)
    logits_ref[0:HALF_T, :] = l0
    mask_ref[0:HALF_T, :] = k0

    l1, k1, p1, u1 = _half(x_ref[HALF_T:BLOCK_T, :], w2, wh)
    logits_ref[HALF_T:BLOCK_T, :] = l1
    mask_ref[HALF_T:BLOCK_T, :] = k1

    psum = p0 + p1
    usum = u0 + u1

    @pl.when(i == 0)
    def _init():
        psum_ref[...] = psum
        usum_ref[...] = usum

    @pl.when(i > 0)
    def _acc():
        psum_ref[...] = psum_ref[...] + psum
        usum_ref[...] = usum_ref[...] + usum

    @pl.when(i == N_STEPS - 1)
    def _finalize():
        rp = psum_ref[...] / TOKENS
        us = usum_ref[...] / TOKENS
        mm = jnp.max(rp)
        lse = jnp.log(jnp.sum(jnp.exp(rp - mm))) + mm
        logp = rp - lse
        loss_ref[...] = (-jnp.sum(us * logp, axis=1, keepdims=True)
                         * LOAD_BALANCING_LAMBDA)


def kernel(hidden_states, W):
    b, s, h = hidden_states.shape
    x = hidden_states.reshape(b * s, h)
    wh = W.astype(jnp.bfloat16)
    wl = (W - wh.astype(jnp.float32)).astype(jnp.bfloat16)
    w2 = jnp.concatenate([wh, wl], axis=0)
    logits, mask, loss = pl.pallas_call(
        _router_kernel,
        grid=(N_STEPS,),
        in_specs=[
            pl.BlockSpec((BLOCK_T, HIDDEN), lambda i: (i, 0)),
            pl.BlockSpec((2 * NUM_EXPERTS, HIDDEN), lambda i: (0, 0)),
            pl.BlockSpec((NUM_EXPERTS, HIDDEN), lambda i: (0, 0)),
        ],
        out_specs=[
            pl.BlockSpec((BLOCK_T, NUM_EXPERTS), lambda i: (i, 0)),
            pl.BlockSpec((BLOCK_T, NUM_EXPERTS), lambda i: (i, 0)),
            pl.BlockSpec((1, 1), lambda i: (0, 0)),
        ],
        out_shape=[
            jax.ShapeDtypeStruct((TOKENS, NUM_EXPERTS), jnp.float32),
            jax.ShapeDtypeStruct((TOKENS, NUM_EXPERTS), jnp.float32),
            jax.ShapeDtypeStruct((1, 1), jnp.float32),
        ],
        scratch_shapes=[
            pltpu.VMEM((1, NUM_EXPERTS), jnp.float32),
            pltpu.VMEM((1, NUM_EXPERTS), jnp.float32),
        ],
        compiler_params=pltpu.CompilerParams(
            dimension_semantics=("arbitrary",),
            vmem_limit_bytes=100 * 1024 * 1024),
    )(x, w2, wh)
    return (logits.reshape(b, s, NUM_EXPERTS),
            mask.reshape(b, s, NUM_EXPERTS),
            loss[0, 0])
